# baseline (speedup 1.0000x reference)
import functools
import math

import jax
import jax.numpy as jnp
from jax import lax
from jax.experimental import pallas as pl
from jax.experimental.pallas import tpu as pltpu
from jax.experimental.pallas import tpu_sc as plsc

EMBED = 128
NC = 2
NS = 16
NW = NC * NS


def _transform_kernel(nr, ne, tr, te, wr, br, gr, betar, we, be, ge, betae,
                      outr, oute):
    inv_sqrt2 = 0.7071067811865476

    def tfm(x, W, b, g, beta, nvalid):
        y = lax.dot_general(x, W, (((1,), (1,)), ((), ())),
                            preferred_element_type=jnp.float32)
        y = y + b
        mu = jnp.mean(y, axis=-1, keepdims=True)
        var = jnp.mean((y - mu) ** 2, axis=-1, keepdims=True)
        y = (y - mu) / jnp.sqrt(var + 1e-5) * g + beta
        y = y * 0.5 * (1.0 + lax.erf(y * inv_sqrt2))
        rows = lax.broadcasted_iota(jnp.int32, y.shape, 0)
        return jnp.where(rows < nvalid, y, 0.0)

    outr[...] = tfm(tr[...], wr[...], br[...], gr[...], betar[...], nr)
    oute[...] = tfm(te[...], we[...], be[...], ge[...], betae[...], ne)


def _transform_tables(tr, te, Wr, br, gr, betar, We, be, ge, betae):
    nr, ne = tr.shape[0], te.shape[0]
    nr_pad = 8 * math.ceil((nr + 1) / 8)
    ne_pad = 8 * math.ceil((ne + 1) / 8)
    tr_p = jnp.zeros((nr_pad, EMBED), jnp.float32).at[:nr].set(tr)
    te_p = jnp.zeros((ne_pad, EMBED), jnp.float32).at[:ne].set(te)
    r2 = lambda v: v.reshape(1, EMBED)
    outr, oute = pl.pallas_call(
        functools.partial(_transform_kernel, nr, ne),
        out_shape=(jax.ShapeDtypeStruct((nr_pad, EMBED), jnp.float32),
                   jax.ShapeDtypeStruct((ne_pad, EMBED), jnp.float32)),
    )(tr_p, te_p, Wr, r2(br), r2(gr), r2(betar), We, r2(be), r2(ge), r2(betae))
    return outr, oute


def _sc_body(n_per_b, chunks_per_w, zrow_r, zrow_e,
             tabr, tabe, rid2, eid2, lensr2, regst, entst,
             ridc, eidc, lensc, regrows, entrows, sem):
    wid = lax.axis_index("s") * NC + lax.axis_index("c")
    crow0 = wid * chunks_per_w
    base = wid * chunks_per_w * 128

    def chunk(ci, carry):
        tok0 = pl.multiple_of(base + ci * 128, 128)
        pltpu.sync_copy(rid2.at[crow0 + ci], ridc)
        pltpu.sync_copy(eid2.at[crow0 + ci], eidc)
        pltpu.sync_copy(lensr2.at[crow0 + ci], lensc)
        for v in range(8):
            tloc = ci * 128 + v * 16 + lax.iota(jnp.int32, 16)
            n = tloc % n_per_b
            lv = lensc[pl.ds(v * 16, 16)]
            m = n < lv
            rv = ridc[pl.ds(v * 16, 16)]
            ridc[pl.ds(v * 16, 16)] = jnp.where(m, rv, zrow_r)
            ev = eidc[pl.ds(v * 16, 16)]
            eidc[pl.ds(v * 16, 16)] = jnp.where(m, ev, zrow_e)
        pltpu.async_copy(tabr.at[ridc], regrows, sem).wait()
        pltpu.async_copy(tabe.at[eidc], entrows, sem).wait()
        pltpu.sync_copy(regrows, regst.at[pl.ds(tok0, 128)])
        pltpu.sync_copy(entrows, entst.at[pl.ds(tok0, 128)])
        return carry

    lax.fori_loop(0, chunks_per_w, chunk, 0)


def _assemble_kernel(bb_ref, lens_ref, reg_ref, ent_ref, feat_ref, mask_ref):
    bb = bb_ref[...]
    x1 = bb[:, :, 0]
    y1 = bb[:, :, 1]
    x2 = bb[:, :, 2]
    y2 = bb[:, :, 3]
    w = x2 - x1
    h = y2 - y1
    bbf = jnp.stack([x1, y1, x2, y2, w * h, w / (h + 1e-6)], axis=-1)
    n_iota = lax.broadcasted_iota(jnp.int32, bb.shape[:2], 1)
    mask = (n_iota < lens_ref[...]).astype(jnp.float32)
    mask_ref[...] = mask
    bbf = bbf * mask[:, :, None]
    feat_ref[...] = jnp.concatenate([bbf, reg_ref[...], ent_ref[...]],
                                    axis=-1)


def _assemble(bboxes, lengths, regst, entst, b_blk):
    B, N = bboxes.shape[:2]
    out_d = 6 + 2 * EMBED
    grid = (B // b_blk,)
    return pl.pallas_call(
        _assemble_kernel,
        grid=grid,
        in_specs=[
            pl.BlockSpec((b_blk, N, 4), lambda i: (i, 0, 0)),
            pl.BlockSpec((b_blk, 1), lambda i: (i, 0)),
            pl.BlockSpec((b_blk, N, EMBED), lambda i: (i, 0, 0)),
            pl.BlockSpec((b_blk, N, EMBED), lambda i: (i, 0, 0)),
        ],
        out_specs=(
            pl.BlockSpec((b_blk, N, out_d), lambda i: (i, 0, 0)),
            pl.BlockSpec((b_blk, N), lambda i: (i, 0)),
        ),
        out_shape=(jax.ShapeDtypeStruct((B, N, out_d), jnp.float32),
                   jax.ShapeDtypeStruct((B, N), jnp.float32)),
    )(bboxes, lengths.reshape(B, 1), regst.reshape(B, N, EMBED),
      entst.reshape(B, N, EMBED))


def kernel(bboxes, region_ids, entity_ids, lengths, region_table, entity_table,
           Wr, br, gr, betar, We, be, ge, betae):
    B, N = region_ids.shape
    tokens = B * N
    assert tokens % (NW * 128) == 0 and (tokens // NW) % N == 0
    chunks_per_w = tokens // (NW * 128)

    tabr, tabe = _transform_tables(region_table, entity_table,
                                   Wr, br, gr, betar, We, be, ge, betae)
    zrow_r = region_table.shape[0]
    zrow_e = entity_table.shape[0]

    rid2 = region_ids.astype(jnp.int32).reshape(tokens // 128, 128)
    eid2 = entity_ids.astype(jnp.int32).reshape(tokens // 128, 128)
    lensr2 = jnp.broadcast_to(lengths.astype(jnp.int32)[:, None],
                              (B, N)).reshape(tokens // 128, 128)

    mesh = plsc.VectorSubcoreMesh(core_axis_name="c", subcore_axis_name="s")
    regst, entst = pl.kernel(
        functools.partial(_sc_body, N, chunks_per_w, zrow_r, zrow_e),
        out_type=(jax.ShapeDtypeStruct((tokens, EMBED), jnp.float32),
                  jax.ShapeDtypeStruct((tokens, EMBED), jnp.float32)),
        mesh=mesh,
        compiler_params=pltpu.CompilerParams(use_tc_tiling_on_sc=False),
        scratch_types=(
            pltpu.VMEM((128,), jnp.int32),
            pltpu.VMEM((128,), jnp.int32),
            pltpu.VMEM((128,), jnp.int32),
            pltpu.VMEM((128, EMBED), jnp.float32),
            pltpu.VMEM((128, EMBED), jnp.float32),
            pltpu.SemaphoreType.DMA,
        ),
    )(tabr, tabe, rid2, eid2, lensr2)

    feat, mask = _assemble(bboxes, lengths.astype(jnp.int32), regst, entst,
                           b_blk=64)
    return feat, mask
